# baseline (speedup 1.0000x reference)
import functools

import jax
import jax.numpy as jnp
from jax import lax
from jax.experimental import pallas as pl
from jax.experimental.pallas import tpu as pltpu
from jax.experimental.pallas import tpu_sc as plsc

N_NODES = 10000
N_EDGES = 320000
D = 128
NUM_CLASSES = 40

NC = 2
NS = 16
NW = NC * NS

EPT = N_EDGES // NW
EPT_P = 10240
MCHUNK = 80
MGRP = 16
MGRPS = EPT_P // (MCHUNK * MGRP)

CCHUNK = 2000
CCH = EPT // CCHUNK

RPT = 624
RTAIL = N_NODES - RPT * NS
NPAD = 10240
DRPT = NPAD // NS
DCK = DRPT // 128

_mesh = plsc.VectorSubcoreMesh(
    core_axis_name="c", subcore_axis_name="s", num_cores=NC, num_subcores=NS)
_params = pltpu.CompilerParams(needs_layout_passes=False)


def _rsqrt16(x):
  x = jnp.maximum(x, 1.0)
  i = lax.bitcast_convert_type(x, jnp.int32)
  i = jnp.int32(0x5F3759DF) - lax.shift_right_logical(i, 1)
  y = lax.bitcast_convert_type(i, jnp.float32)
  for _ in range(3):
    y = y * (1.5 - 0.5 * x * y * y)
  return y


@functools.partial(
    pl.kernel,
    out_type=(
        jax.ShapeDtypeStruct((NW, 1, NPAD), jnp.float32),
        jax.ShapeDtypeStruct((NW, 1, NPAD), jnp.float32),
    ),
    mesh=_mesh,
    compiler_params=_params,
    scratch_types=[
        pltpu.VMEM((NPAD,), jnp.float32),
        pltpu.VMEM((NPAD,), jnp.float32),
        pltpu.VMEM((1, CCHUNK), jnp.int32),
        pltpu.VMEM((1, CCHUNK), jnp.int32),
    ],
)
def _hist(src_c, dst_c, hs_hbm, hd_hbm, ho_v, hi_v, e1_v, e2_v):
  cid = lax.axis_index("c")
  sid = lax.axis_index("s")
  wid = cid * NS + sid
  zeros16 = jnp.zeros((16,), jnp.float32)
  ones16 = jnp.ones((16,), jnp.float32)

  @pl.loop(0, NPAD // 16)
  def _(g):
    ho_v[pl.ds(g * 16, 16)] = zeros16
    hi_v[pl.ds(g * 16, 16)] = zeros16

  @pl.loop(0, CCH)
  def _(cnk):
    pltpu.sync_copy(src_c.at[wid, cnk], e1_v)
    pltpu.sync_copy(dst_c.at[wid, cnk], e2_v)

    @pl.loop(0, CCHUNK // 16)
    def _(j):
      s = e1_v[0, pl.ds(j * 16, 16)]
      plsc.addupdate_scatter(ho_v, [s], ones16)
      d = e2_v[0, pl.ds(j * 16, 16)]
      plsc.addupdate_scatter(hi_v, [d], ones16)

  pltpu.sync_copy(ho_v, hs_hbm.at[wid, 0])
  pltpu.sync_copy(hi_v, hd_hbm.at[wid, 0])


@functools.partial(
    pl.kernel,
    out_type=(
        jax.ShapeDtypeStruct((1, NPAD), jnp.float32),
        jax.ShapeDtypeStruct((1, NPAD), jnp.float32),
    ),
    mesh=_mesh,
    compiler_params=_params,
    scratch_types=[
        pltpu.VMEM((DRPT,), jnp.float32),
        pltpu.VMEM((DRPT,), jnp.float32),
        pltpu.VMEM((DRPT,), jnp.float32),
    ],
)
def _norm(hs_hbm, hd_hbm, ns_hbm, nd_hbm, deg_v, tmp_v, ninv_v):
  cid = lax.axis_index("c")
  sid = lax.axis_index("s")
  rbase = sid * DRPT
  zeros16 = jnp.zeros((16,), jnp.float32)

  @pl.when(cid == 0)
  def _():
    def _one(h_hbm, n_hbm):
      @pl.loop(0, DRPT // 16)
      def _(g):
        deg_v[pl.ds(g * 16, 16)] = zeros16

      for w in range(NW):
        pltpu.sync_copy(h_hbm.at[w, 0, pl.ds(rbase, DRPT)], tmp_v)

        @pl.loop(0, DRPT // 16)
        def _(g):
          deg_v[pl.ds(g * 16, 16)] = (deg_v[pl.ds(g * 16, 16)]
                                      + tmp_v[pl.ds(g * 16, 16)])

      @pl.loop(0, DRPT // 16)
      def _(g):
        ninv_v[pl.ds(g * 16, 16)] = _rsqrt16(deg_v[pl.ds(g * 16, 16)])

      pltpu.sync_copy(ninv_v, n_hbm.at[0, pl.ds(rbase, DRPT)])

    _one(hs_hbm, ns_hbm)
    _one(hd_hbm, nd_hbm)


@functools.partial(
    pl.kernel,
    out_type=jax.ShapeDtypeStruct((NW, CCH, 1, CCHUNK), jnp.float32),
    mesh=_mesh,
    compiler_params=_params,
    scratch_types=[
        pltpu.VMEM((NPAD,), jnp.float32),
        pltpu.VMEM((NPAD,), jnp.float32),
        pltpu.VMEM((1, CCHUNK), jnp.int32),
        pltpu.VMEM((1, CCHUNK), jnp.int32),
        pltpu.VMEM((1, CCHUNK), jnp.float32),
        pltpu.VMEM((1, CCHUNK), jnp.float32),
    ],
)
def _coef(src_c, dst_c, ew_c, ns_hbm, nd_hbm, coef_out,
          ns_full, nd_full, e1_v, e2_v, ew_v, co_v):
  cid = lax.axis_index("c")
  sid = lax.axis_index("s")
  wid = cid * NS + sid

  pltpu.sync_copy(ns_hbm.at[0], ns_full)
  pltpu.sync_copy(nd_hbm.at[0], nd_full)

  @pl.loop(0, CCH)
  def _(cnk):
    pltpu.sync_copy(src_c.at[wid, cnk], e1_v)
    pltpu.sync_copy(dst_c.at[wid, cnk], e2_v)
    pltpu.sync_copy(ew_c.at[wid, cnk], ew_v)

    @pl.loop(0, CCHUNK // 16)
    def _(j):
      s = e1_v[0, pl.ds(j * 16, 16)]
      d = e2_v[0, pl.ds(j * 16, 16)]
      c = (ew_v[0, pl.ds(j * 16, 16)]
           * plsc.load_gather(ns_full, [s])
           * plsc.load_gather(nd_full, [d]))
      co_v[0, pl.ds(j * 16, 16)] = c

    pltpu.sync_copy(co_v, coef_out.at[wid, cnk])


@functools.partial(
    pl.kernel,
    out_type=jax.ShapeDtypeStruct((NC * N_NODES, D), jnp.float32),
    mesh=_mesh,
    compiler_params=_params,
    scratch_types=[
        pltpu.VMEM((MGRP, MCHUNK), jnp.int32),
        pltpu.VMEM((MGRP, MCHUNK), jnp.int32),
        pltpu.VMEM((MGRP, MCHUNK), jnp.float32),
        pltpu.VMEM((MCHUNK, D), jnp.float32),
        pltpu.VMEM((MCHUNK, D), jnp.float32),
        pltpu.VMEM((MCHUNK, D), jnp.float32),
        pltpu.VMEM((MCHUNK, D), jnp.float32),
        pltpu.VMEM_SHARED((N_NODES, D), jnp.float32),
        pltpu.SemaphoreType.DMA,
        pltpu.SemaphoreType.DMA,
        pltpu.SemaphoreType.DMA,
        pltpu.SemaphoreType.DMA,
    ],
)
def _msg(h_hbm, src4, dst4, coef4, part_out,
         srcv, dstv, coefv, rows0, rows1, rows2, rows3,
         agg_sh, gsem0, gsem1, gsem2, gsem3):
  cid = lax.axis_index("c")
  sid = lax.axis_index("s")
  wid = cid * NS + sid
  zeros16 = jnp.zeros((16,), jnp.float32)
  rows_bufs = (rows0, rows1, rows2, rows3)
  gsems = (gsem0, gsem1, gsem2, gsem3)

  @pl.loop(0, MCHUNK)
  def _(r):
    for j in range(D // 16):
      rows0[r, pl.ds(j * 16, 16)] = zeros16

  rbase = sid * RPT
  for i in range(RPT // MCHUNK):
    pltpu.sync_copy(rows0, agg_sh.at[pl.ds(rbase + i * MCHUNK, MCHUNK)])
  rem = RPT - (RPT // MCHUNK) * MCHUNK
  if rem:
    pltpu.sync_copy(rows0.at[pl.ds(0, rem)],
                    agg_sh.at[pl.ds(rbase + RPT - rem, rem)])

  @pl.when(sid == NS - 1)
  def _():
    pltpu.sync_copy(rows0.at[pl.ds(0, RTAIL)],
                    agg_sh.at[pl.ds(RPT * NS, RTAIL)])

  plsc.subcore_barrier()

  def _scale(rows, cnk):
    for g in range(MCHUNK // 16):
      c16 = coefv[cnk, pl.ds(g * 16, 16)]
      for l in range(16):
        e = g * 16 + l
        cval = c16[l]
        for j in range(D // 16):
          rows[e, pl.ds(j * 16, 16)] = rows[e, pl.ds(j * 16, 16)] * cval

  @pl.loop(0, MGRPS)
  def _(grp):
    pltpu.sync_copy(src4.at[wid, grp], srcv)
    pltpu.sync_copy(dst4.at[wid, grp], dstv)
    pltpu.sync_copy(coef4.at[wid, grp], coefv)

    @pl.loop(0, MGRP // 4)
    def _(q):
      c0 = 4 * q
      descs = [
          pltpu.async_copy(h_hbm.at[srcv.at[c0 + k]], rows_bufs[k], gsems[k])
          for k in range(4)
      ]
      for k in range(4):
        descs[k].wait()
        _scale(rows_bufs[k], c0 + k)
        pltpu.sync_copy(rows_bufs[k], agg_sh.at[dstv.at[c0 + k]], add=True)

  plsc.subcore_barrier()

  obase = cid * N_NODES + rbase
  pltpu.sync_copy(agg_sh.at[pl.ds(rbase, RPT)], part_out.at[pl.ds(obase, RPT)])

  @pl.when(sid == NS - 1)
  def _():
    pltpu.sync_copy(agg_sh.at[pl.ds(RPT * NS, RTAIL)],
                    part_out.at[pl.ds(cid * N_NODES + RPT * NS, RTAIL)])


BM = 1000
_GRID = N_NODES // BM


def _first_body(x_ref, w1_ref, wres_ref, bres_ref, h1_ref, res_ref):
  x = x_ref[...]
  h1_ref[...] = jnp.dot(x, w1_ref[...], preferred_element_type=jnp.float32)
  res_ref[...] = (jnp.dot(x, wres_ref[...], preferred_element_type=jnp.float32)
                  + bres_ref[...])


_first = pl.pallas_call(
    _first_body,
    grid=(_GRID,),
    in_specs=[
        pl.BlockSpec((BM, D), lambda i: (i, 0)),
        pl.BlockSpec((D, D), lambda i: (0, 0)),
        pl.BlockSpec((D, D), lambda i: (0, 0)),
        pl.BlockSpec((1, D), lambda i: (0, 0)),
    ],
    out_specs=[pl.BlockSpec((BM, D), lambda i: (i, 0))] * 2,
    out_shape=[jax.ShapeDtypeStruct((N_NODES, D), jnp.float32)] * 2,
)


def _next_body(p0_ref, p1_ref, b_ref, w_ref, o_ref):
  y = jnp.maximum(p0_ref[...] + p1_ref[...] + b_ref[...], 0.0)
  o_ref[...] = jnp.dot(y, w_ref[...], preferred_element_type=jnp.float32)


_next = pl.pallas_call(
    _next_body,
    grid=(_GRID,),
    in_specs=[
        pl.BlockSpec((BM, D), lambda i: (i, 0)),
        pl.BlockSpec((BM, D), lambda i: (i + _GRID, 0)),
        pl.BlockSpec((1, D), lambda i: (0, 0)),
        pl.BlockSpec((D, D), lambda i: (0, 0)),
    ],
    out_specs=pl.BlockSpec((BM, D), lambda i: (i, 0)),
    out_shape=jax.ShapeDtypeStruct((N_NODES, D), jnp.float32),
)


def _final_body(p0_ref, p1_ref, b_ref, res_ref, wop_ref, bop_ref, o_ref):
  y = jnp.maximum(p0_ref[...] + p1_ref[...] + b_ref[...] + res_ref[...], 0.0)
  o_ref[...] = (jnp.dot(y, wop_ref[...], preferred_element_type=jnp.float32)
                + bop_ref[...])


_final = pl.pallas_call(
    _final_body,
    grid=(_GRID,),
    in_specs=[
        pl.BlockSpec((BM, D), lambda i: (i, 0)),
        pl.BlockSpec((BM, D), lambda i: (i + _GRID, 0)),
        pl.BlockSpec((1, D), lambda i: (0, 0)),
        pl.BlockSpec((BM, D), lambda i: (i, 0)),
        pl.BlockSpec((D, NUM_CLASSES), lambda i: (0, 0)),
        pl.BlockSpec((1, NUM_CLASSES), lambda i: (0, 0)),
    ],
    out_specs=pl.BlockSpec((BM, NUM_CLASSES), lambda i: (i, 0)),
    out_shape=jax.ShapeDtypeStruct((N_NODES, NUM_CLASSES), jnp.float32),
)


def kernel(inputs, edge_index, edge_weight,
           W1, b1, W2, b2, W3, b3, W4, b4, W_res, b_res, W_op, b_op):
  ei = edge_index.astype(jnp.int32)
  src = ei[0]
  dst = ei[1]
  src_c = src.reshape(NW, CCH, 1, CCHUNK)
  dst_c = dst.reshape(NW, CCH, 1, CCHUNK)
  ew_c = edge_weight.reshape(NW, CCH, 1, CCHUNK)
  hs, hd = _hist(src_c, dst_c)
  ns, nd = _norm(hs, hd)
  coef = _coef(src_c, dst_c, ew_c, ns, nd)
  pad_i = jnp.zeros((NW, EPT_P - EPT), jnp.int32)
  pad_f = jnp.zeros((NW, EPT_P - EPT), jnp.float32)
  src4 = jnp.concatenate([src.reshape(NW, EPT), pad_i], 1).reshape(
      NW, MGRPS, MGRP, MCHUNK)
  dst4 = jnp.concatenate([dst.reshape(NW, EPT), pad_i], 1).reshape(
      NW, MGRPS, MGRP, MCHUNK)
  coef4 = jnp.concatenate([coef.reshape(NW, EPT), pad_f], 1).reshape(
      NW, MGRPS, MGRP, MCHUNK)

  h1, res = _first(inputs, W1, W_res, b_res.reshape(1, D))
  parts = _msg(h1, src4, dst4, coef4)
  h2 = _next(parts, parts, b1.reshape(1, D), W2)
  parts = _msg(h2, src4, dst4, coef4)
  h3 = _next(parts, parts, b2.reshape(1, D), W3)
  parts = _msg(h3, src4, dst4, coef4)
  h4 = _next(parts, parts, b3.reshape(1, D), W4)
  parts = _msg(h4, src4, dst4, coef4)
  out = _final(parts, parts, b4.reshape(1, D), res,
               W_op, b_op.reshape(1, NUM_CLASSES))
  return out


# speedup vs baseline: 1.8830x; 1.8830x over previous
import functools

import jax
import jax.numpy as jnp
from jax import lax
from jax.experimental import pallas as pl
from jax.experimental.pallas import tpu as pltpu
from jax.experimental.pallas import tpu_sc as plsc

N_NODES = 10000
N_EDGES = 320000
D = 128
NUM_CLASSES = 40

NC = 2
NS = 16
NW = NC * NS

EPT = N_EDGES // NW
MCHUNK = 80
MGRP = 25
MGRPS = EPT // (MCHUNK * MGRP)

CCHUNK = 2000
CCH = EPT // CCHUNK

RPT = 624
RTAIL = N_NODES - RPT * NS
NPAD = 10240
DRPT = NPAD // NS
DCK = DRPT // 128

_mesh = plsc.VectorSubcoreMesh(
    core_axis_name="c", subcore_axis_name="s", num_cores=NC, num_subcores=NS)
_params = pltpu.CompilerParams(needs_layout_passes=False)


def _rsqrt16(x):
  x = jnp.maximum(x, 1.0)
  i = lax.bitcast_convert_type(x, jnp.int32)
  i = jnp.int32(0x5F3759DF) - lax.shift_right_logical(i, 1)
  y = lax.bitcast_convert_type(i, jnp.float32)
  for _ in range(3):
    y = y * (1.5 - 0.5 * x * y * y)
  return y


@functools.partial(
    pl.kernel,
    out_type=(
        jax.ShapeDtypeStruct((NW, 1, NPAD), jnp.float32),
        jax.ShapeDtypeStruct((NW, 1, NPAD), jnp.float32),
    ),
    mesh=_mesh,
    compiler_params=_params,
    scratch_types=[
        pltpu.VMEM((NPAD,), jnp.float32),
        pltpu.VMEM((NPAD,), jnp.float32),
        pltpu.VMEM((1, CCHUNK), jnp.int32),
        pltpu.VMEM((1, CCHUNK), jnp.int32),
    ],
)
def _hist(src_c, dst_c, hs_hbm, hd_hbm, ho_v, hi_v, e1_v, e2_v):
  cid = lax.axis_index("c")
  sid = lax.axis_index("s")
  wid = cid * NS + sid
  zeros16 = jnp.zeros((16,), jnp.float32)
  ones16 = jnp.ones((16,), jnp.float32)

  @pl.loop(0, NPAD // 16)
  def _(g):
    ho_v[pl.ds(g * 16, 16)] = zeros16
    hi_v[pl.ds(g * 16, 16)] = zeros16

  @pl.loop(0, CCH)
  def _(cnk):
    pltpu.sync_copy(src_c.at[wid, cnk], e1_v)
    pltpu.sync_copy(dst_c.at[wid, cnk], e2_v)

    @pl.loop(0, CCHUNK // 16)
    def _(j):
      s = e1_v[0, pl.ds(j * 16, 16)]
      plsc.addupdate_scatter(ho_v, [s], ones16)
      d = e2_v[0, pl.ds(j * 16, 16)]
      plsc.addupdate_scatter(hi_v, [d], ones16)

  pltpu.sync_copy(ho_v, hs_hbm.at[wid, 0])
  pltpu.sync_copy(hi_v, hd_hbm.at[wid, 0])


@functools.partial(
    pl.kernel,
    out_type=(
        jax.ShapeDtypeStruct((1, NPAD), jnp.float32),
        jax.ShapeDtypeStruct((1, NPAD), jnp.float32),
    ),
    mesh=_mesh,
    compiler_params=_params,
    scratch_types=[
        pltpu.VMEM((DRPT,), jnp.float32),
        pltpu.VMEM((DRPT,), jnp.float32),
        pltpu.VMEM((DRPT,), jnp.float32),
    ],
)
def _norm(hs_hbm, hd_hbm, ns_hbm, nd_hbm, deg_v, tmp_v, ninv_v):
  cid = lax.axis_index("c")
  sid = lax.axis_index("s")
  rbase = sid * DRPT
  zeros16 = jnp.zeros((16,), jnp.float32)

  @pl.when(cid == 0)
  def _():
    def _one(h_hbm, n_hbm):
      @pl.loop(0, DRPT // 16)
      def _(g):
        deg_v[pl.ds(g * 16, 16)] = zeros16

      for w in range(NW):
        pltpu.sync_copy(h_hbm.at[w, 0, pl.ds(rbase, DRPT)], tmp_v)

        @pl.loop(0, DRPT // 16)
        def _(g):
          deg_v[pl.ds(g * 16, 16)] = (deg_v[pl.ds(g * 16, 16)]
                                      + tmp_v[pl.ds(g * 16, 16)])

      @pl.loop(0, DRPT // 16)
      def _(g):
        ninv_v[pl.ds(g * 16, 16)] = _rsqrt16(deg_v[pl.ds(g * 16, 16)])

      pltpu.sync_copy(ninv_v, n_hbm.at[0, pl.ds(rbase, DRPT)])

    _one(hs_hbm, ns_hbm)
    _one(hd_hbm, nd_hbm)


@functools.partial(
    pl.kernel,
    out_type=jax.ShapeDtypeStruct((NW, CCH, 1, CCHUNK), jnp.float32),
    mesh=_mesh,
    compiler_params=_params,
    scratch_types=[
        pltpu.VMEM((NPAD,), jnp.float32),
        pltpu.VMEM((NPAD,), jnp.float32),
        pltpu.VMEM((1, CCHUNK), jnp.int32),
        pltpu.VMEM((1, CCHUNK), jnp.int32),
        pltpu.VMEM((1, CCHUNK), jnp.float32),
        pltpu.VMEM((1, CCHUNK), jnp.float32),
    ],
)
def _coef(src_c, dst_c, ew_c, ns_hbm, nd_hbm, coef_out,
          ns_full, nd_full, e1_v, e2_v, ew_v, co_v):
  cid = lax.axis_index("c")
  sid = lax.axis_index("s")
  wid = cid * NS + sid

  pltpu.sync_copy(ns_hbm.at[0], ns_full)
  pltpu.sync_copy(nd_hbm.at[0], nd_full)

  @pl.loop(0, CCH)
  def _(cnk):
    pltpu.sync_copy(src_c.at[wid, cnk], e1_v)
    pltpu.sync_copy(dst_c.at[wid, cnk], e2_v)
    pltpu.sync_copy(ew_c.at[wid, cnk], ew_v)

    @pl.loop(0, CCHUNK // 16)
    def _(j):
      s = e1_v[0, pl.ds(j * 16, 16)]
      d = e2_v[0, pl.ds(j * 16, 16)]
      c = (ew_v[0, pl.ds(j * 16, 16)]
           * plsc.load_gather(ns_full, [s])
           * plsc.load_gather(nd_full, [d]))
      co_v[0, pl.ds(j * 16, 16)] = c

    pltpu.sync_copy(co_v, coef_out.at[wid, cnk])


@functools.partial(
    pl.kernel,
    out_type=jax.ShapeDtypeStruct((NC * N_NODES, D), jnp.float32),
    mesh=_mesh,
    compiler_params=_params,
    scratch_types=[
        pltpu.VMEM((MGRP, MCHUNK), jnp.int32),
        pltpu.VMEM((MGRP, MCHUNK), jnp.int32),
        pltpu.VMEM((MGRP, MCHUNK), jnp.float32),
        pltpu.VMEM((MCHUNK, D), jnp.float32),
        pltpu.VMEM_SHARED((N_NODES, D), jnp.float32),
        pltpu.SemaphoreType.DMA,
    ],
)
def _msg(h_hbm, src4, dst4, coef4, part_out,
         srcv, dstv, coefv, rows_v, agg_sh, sem):
  cid = lax.axis_index("c")
  sid = lax.axis_index("s")
  wid = cid * NS + sid
  zeros16 = jnp.zeros((16,), jnp.float32)

  @pl.loop(0, MCHUNK)
  def _(r):
    for j in range(D // 16):
      rows_v[r, pl.ds(j * 16, 16)] = zeros16

  rbase = sid * RPT
  for i in range(RPT // MCHUNK):
    pltpu.sync_copy(rows_v, agg_sh.at[pl.ds(rbase + i * MCHUNK, MCHUNK)])
  rem = RPT - (RPT // MCHUNK) * MCHUNK
  if rem:
    pltpu.sync_copy(rows_v.at[pl.ds(0, rem)],
                    agg_sh.at[pl.ds(rbase + RPT - rem, rem)])

  @pl.when(sid == NS - 1)
  def _():
    pltpu.sync_copy(rows_v.at[pl.ds(0, RTAIL)],
                    agg_sh.at[pl.ds(RPT * NS, RTAIL)])

  plsc.subcore_barrier()

  @pl.loop(0, MGRPS)
  def _(grp):
    pltpu.sync_copy(src4.at[wid, grp], srcv)
    pltpu.sync_copy(dst4.at[wid, grp], dstv)
    pltpu.sync_copy(coef4.at[wid, grp], coefv)

    @pl.loop(0, MGRP)
    def _(cnk):
      pltpu.async_copy(h_hbm.at[srcv.at[cnk]], rows_v, sem).wait()

      for g in range(MCHUNK // 16):
        c16 = coefv[cnk, pl.ds(g * 16, 16)]
        for l in range(16):
          e = g * 16 + l
          cval = c16[l]
          for j in range(D // 16):
            rows_v[e, pl.ds(j * 16, 16)] = rows_v[e, pl.ds(j * 16, 16)] * cval

      pltpu.sync_copy(rows_v, agg_sh.at[dstv.at[cnk]], add=True)

  plsc.subcore_barrier()

  obase = cid * N_NODES + rbase
  pltpu.sync_copy(agg_sh.at[pl.ds(rbase, RPT)], part_out.at[pl.ds(obase, RPT)])

  @pl.when(sid == NS - 1)
  def _():
    pltpu.sync_copy(agg_sh.at[pl.ds(RPT * NS, RTAIL)],
                    part_out.at[pl.ds(cid * N_NODES + RPT * NS, RTAIL)])


BM = 1000
_GRID = N_NODES // BM


def _first_body(x_ref, w1_ref, wres_ref, bres_ref, h1_ref, res_ref):
  x = x_ref[...]
  h1_ref[...] = jnp.dot(x, w1_ref[...], preferred_element_type=jnp.float32)
  res_ref[...] = (jnp.dot(x, wres_ref[...], preferred_element_type=jnp.float32)
                  + bres_ref[...])


_first = pl.pallas_call(
    _first_body,
    grid=(_GRID,),
    in_specs=[
        pl.BlockSpec((BM, D), lambda i: (i, 0)),
        pl.BlockSpec((D, D), lambda i: (0, 0)),
        pl.BlockSpec((D, D), lambda i: (0, 0)),
        pl.BlockSpec((1, D), lambda i: (0, 0)),
    ],
    out_specs=[pl.BlockSpec((BM, D), lambda i: (i, 0))] * 2,
    out_shape=[jax.ShapeDtypeStruct((N_NODES, D), jnp.float32)] * 2,
)


def _next_body(p0_ref, p1_ref, b_ref, w_ref, o_ref):
  y = jnp.maximum(p0_ref[...] + p1_ref[...] + b_ref[...], 0.0)
  o_ref[...] = jnp.dot(y, w_ref[...], preferred_element_type=jnp.float32)


_next = pl.pallas_call(
    _next_body,
    grid=(_GRID,),
    in_specs=[
        pl.BlockSpec((BM, D), lambda i: (i, 0)),
        pl.BlockSpec((BM, D), lambda i: (i + _GRID, 0)),
        pl.BlockSpec((1, D), lambda i: (0, 0)),
        pl.BlockSpec((D, D), lambda i: (0, 0)),
    ],
    out_specs=pl.BlockSpec((BM, D), lambda i: (i, 0)),
    out_shape=jax.ShapeDtypeStruct((N_NODES, D), jnp.float32),
)


def _final_body(p0_ref, p1_ref, b_ref, res_ref, wop_ref, bop_ref, o_ref):
  y = jnp.maximum(p0_ref[...] + p1_ref[...] + b_ref[...] + res_ref[...], 0.0)
  o_ref[...] = (jnp.dot(y, wop_ref[...], preferred_element_type=jnp.float32)
                + bop_ref[...])


_final = pl.pallas_call(
    _final_body,
    grid=(_GRID,),
    in_specs=[
        pl.BlockSpec((BM, D), lambda i: (i, 0)),
        pl.BlockSpec((BM, D), lambda i: (i + _GRID, 0)),
        pl.BlockSpec((1, D), lambda i: (0, 0)),
        pl.BlockSpec((BM, D), lambda i: (i, 0)),
        pl.BlockSpec((D, NUM_CLASSES), lambda i: (0, 0)),
        pl.BlockSpec((1, NUM_CLASSES), lambda i: (0, 0)),
    ],
    out_specs=pl.BlockSpec((BM, NUM_CLASSES), lambda i: (i, 0)),
    out_shape=jax.ShapeDtypeStruct((N_NODES, NUM_CLASSES), jnp.float32),
)


def kernel(inputs, edge_index, edge_weight,
           W1, b1, W2, b2, W3, b3, W4, b4, W_res, b_res, W_op, b_op):
  ei = edge_index.astype(jnp.int32)
  src = ei[0]
  dst = ei[1]
  src_c = src.reshape(NW, CCH, 1, CCHUNK)
  dst_c = dst.reshape(NW, CCH, 1, CCHUNK)
  ew_c = edge_weight.reshape(NW, CCH, 1, CCHUNK)
  hs, hd = _hist(src_c, dst_c)
  ns, nd = _norm(hs, hd)
  coef = _coef(src_c, dst_c, ew_c, ns, nd)
  src4 = src.reshape(NW, MGRPS, MGRP, MCHUNK)
  dst4 = dst.reshape(NW, MGRPS, MGRP, MCHUNK)
  coef4 = coef.reshape(NW, MGRPS, MGRP, MCHUNK)

  h1, res = _first(inputs, W1, W_res, b_res.reshape(1, D))
  parts = _msg(h1, src4, dst4, coef4)
  h2 = _next(parts, parts, b1.reshape(1, D), W2)
  parts = _msg(h2, src4, dst4, coef4)
  h3 = _next(parts, parts, b2.reshape(1, D), W3)
  parts = _msg(h3, src4, dst4, coef4)
  h4 = _next(parts, parts, b3.reshape(1, D), W4)
  parts = _msg(h4, src4, dst4, coef4)
  out = _final(parts, parts, b4.reshape(1, D), res,
               W_op, b_op.reshape(1, NUM_CLASSES))
  return out
